# Initial kernel scaffold; baseline (speedup 1.0000x reference)
import jax
import jax.numpy as jnp
from jax.experimental import pallas as pl


def kernel(z_e_x, weight):
    raise NotImplementedError("write your pallas kernel here")


# speedup vs baseline: 1.0284x; 1.0284x over previous
import functools

import jax
import jax.numpy as jnp
from jax.experimental import pallas as pl
from jax.experimental.pallas import tpu as pltpu


def _fused_body(w_ref, z_ref, o_ref):
    z = z_ref[0]
    k = z.shape[0]
    m = jnp.max(z, axis=0, keepdims=True)
    iota = jax.lax.broadcasted_iota(jnp.int32, z.shape, 0)
    idx = jnp.min(jnp.where(z == m, iota, k), axis=0, keepdims=True)
    onehot = (iota == idx).astype(w_ref.dtype)
    o_ref[0] = jax.lax.dot_general(
        w_ref[...], onehot, (((0,), (0,)), ((), ())),
        preferred_element_type=jnp.float32,
    )


@jax.jit
def kernel(z_e_x, weight):
    b, k, h, w = z_e_x.shape
    d = weight.shape[1]
    hw = h * w
    z = z_e_x.reshape(b, k, hw)
    out = pl.pallas_call(
        _fused_body,
        grid=(b,),
        in_specs=[
            pl.BlockSpec((k, d), lambda i: (0, 0)),
            pl.BlockSpec((1, k, hw), lambda i: (i, 0, 0)),
        ],
        out_specs=pl.BlockSpec((1, d, hw), lambda i: (i, 0, 0)),
        out_shape=jax.ShapeDtypeStruct((b, d, hw), jnp.float32),
    )(weight, z)
    return out.reshape(b, d, h, w)
